# Initial kernel scaffold; baseline (speedup 1.0000x reference)
import functools

import jax
import jax.numpy as jnp
from jax import lax
from jax.experimental import pallas as pl
from jax.experimental.pallas import tpu as pltpu
from jax.experimental.pallas import tpu_sc as plsc

VOCAB = 1000
TOKENS = 1024 * 50

NC = 2
NS = 16
NW = NC * NS
L = 16
TPW = TOKENS // NW
NB = 32
NBLK = TPW // NB
GPB = NB // L


def _lse_body(t_ref, o_ref):
    t = t_ref[...]
    m = jnp.max(t, axis=1, keepdims=True)
    s = jnp.sum(jnp.exp(t - m), axis=1)
    o_ref[...] = m[:, 0] + jnp.log(s)


def _lse_table(table):
    return pl.pallas_call(
        _lse_body,
        out_shape=jax.ShapeDtypeStruct((VOCAB,), jnp.float32),
    )(table)


def _sc_body(table, idxs, tgts, lse, out, partials,
             idx_v, tgt_v, lse_v, rows_v, acc_v, gsem, osem):
    wid = lax.axis_index("s") * NC + lax.axis_index("c")
    base = pl.multiple_of(wid * TPW, 8)
    pltpu.sync_copy(idxs.at[pl.ds(base, TPW)], idx_v)
    pltpu.sync_copy(tgts.at[pl.ds(base, TPW)], tgt_v)
    pltpu.sync_copy(lse, lse_v)

    def block(b, acc):
        boff = pl.multiple_of(b * NB, 8)
        pltpu.async_copy(table.at[idx_v.at[pl.ds(boff, NB)]], rows_v,
                         gsem).wait()
        cp_out = pltpu.async_copy(
            rows_v, out.at[pl.ds(pl.multiple_of(base + boff, 8), NB)], osem)
        for g in range(GPB):
            off = pl.multiple_of(boff + g * L, 8)
            i16 = idx_v[pl.ds(off, L)]
            t16 = tgt_v[pl.ds(off, L)]
            l16 = plsc.load_gather(lse_v, [i16])
            rid = lax.iota(jnp.int32, L) + g * L
            p16 = plsc.load_gather(rows_v, [rid, t16])
            acc = acc + (l16 - p16)
        cp_out.wait()
        return acc

    acc = lax.fori_loop(0, NBLK, block, jnp.zeros((L,), jnp.float32))
    acc_v[...] = acc
    pltpu.sync_copy(acc_v, partials.at[wid])


def _sc_main(table, flat_idx, flat_tgt, lse):
    mesh = plsc.VectorSubcoreMesh(core_axis_name="c", subcore_axis_name="s")
    f = functools.partial(
        pl.kernel,
        mesh=mesh,
        out_type=[
            jax.ShapeDtypeStruct((TOKENS, VOCAB), jnp.float32),
            jax.ShapeDtypeStruct((NW, L), jnp.float32),
        ],
        scratch_types=[
            pltpu.VMEM((TPW,), jnp.int32),
            pltpu.VMEM((TPW,), jnp.int32),
            pltpu.VMEM((VOCAB,), jnp.float32),
            pltpu.VMEM((NB, VOCAB), jnp.float32),
            pltpu.VMEM((L,), jnp.float32),
            pltpu.SemaphoreType.DMA,
            pltpu.SemaphoreType.DMA,
        ],
    )(_sc_body)
    return f(table, flat_idx, flat_tgt, lse)


def kernel(idx, targets, token_embedding_table):
    flat_idx = idx.reshape(-1).astype(jnp.int32)
    flat_tgt = targets.reshape(-1).astype(jnp.int32)
    lse = _lse_table(token_embedding_table)
    flat_logits, partials = _sc_main(
        token_embedding_table, flat_idx, flat_tgt, lse)
    loss = jnp.sum(partials) / TOKENS
    return (flat_logits, loss)


# speedup vs baseline: 1.3986x; 1.3986x over previous
import functools

import jax
import jax.numpy as jnp
from jax import lax
from jax.experimental import pallas as pl
from jax.experimental.pallas import tpu as pltpu
from jax.experimental.pallas import tpu_sc as plsc

VOCAB = 1000
TOKENS = 1024 * 50

NC = 2
NS = 16
NW = NC * NS
L = 16
TPW = TOKENS // NW
NB = 32
NBLK = TPW // NB
GPB = NB // L


def _lse_body(t_ref, o_ref):
    t = t_ref[...]
    m = jnp.max(t, axis=1, keepdims=True)
    s = jnp.sum(jnp.exp(t - m), axis=1)
    o_ref[...] = m[:, 0] + jnp.log(s)


def _lse_table(table):
    return pl.pallas_call(
        _lse_body,
        out_shape=jax.ShapeDtypeStruct((VOCAB,), jnp.float32),
    )(table)


def _sc_body(table, idxs, tgts, lse, out, partials,
             idx_v, tgt_v, lse_v, rows_v, acc_v, gsem, osem):
    wid = lax.axis_index("s") * NC + lax.axis_index("c")
    base = pl.multiple_of(wid * TPW, 8)
    pltpu.sync_copy(idxs.at[pl.ds(base, TPW)], idx_v)
    pltpu.sync_copy(tgts.at[pl.ds(base, TPW)], tgt_v)
    pltpu.sync_copy(lse, lse_v)

    def block(b, acc):
        boff = pl.multiple_of(b * NB, 8)
        pltpu.async_copy(table.at[idx_v.at[pl.ds(boff, NB)]], rows_v,
                         gsem).wait()
        cp_out = pltpu.async_copy(
            rows_v, out.at[pl.ds(pl.multiple_of(base + boff, 8), NB)], osem)
        for g in range(GPB):
            off = pl.multiple_of(boff + g * L, 8)
            i16 = idx_v[pl.ds(off, L)]
            t16 = tgt_v[pl.ds(off, L)]
            l16 = plsc.load_gather(lse_v, [i16])
            rid = lax.iota(jnp.int32, L) + g * L
            p16 = plsc.load_gather(rows_v, [rid, t16])
            acc = acc + (l16 - p16)
        cp_out.wait()
        return acc

    acc = lax.fori_loop(0, NBLK, block, jnp.zeros((L,), jnp.float32))
    acc_v[...] = acc
    pltpu.sync_copy(acc_v, partials.at[wid])


def _sc_main(table, flat_idx, flat_tgt, lse):
    mesh = plsc.VectorSubcoreMesh(core_axis_name="c", subcore_axis_name="s")
    f = functools.partial(
        pl.kernel,
        mesh=mesh,
        compiler_params=pltpu.CompilerParams(
            needs_layout_passes=False, use_tc_tiling_on_sc=False),
        out_type=[
            jax.ShapeDtypeStruct((TOKENS, VOCAB), jnp.float32),
            jax.ShapeDtypeStruct((NW, L), jnp.float32),
        ],
        scratch_types=[
            pltpu.VMEM((TPW,), jnp.int32),
            pltpu.VMEM((TPW,), jnp.int32),
            pltpu.VMEM((VOCAB,), jnp.float32),
            pltpu.VMEM((NB, VOCAB), jnp.float32),
            pltpu.VMEM((L,), jnp.float32),
            pltpu.SemaphoreType.DMA,
            pltpu.SemaphoreType.DMA,
        ],
    )(_sc_body)
    return f(table, flat_idx, flat_tgt, lse)


def kernel(idx, targets, token_embedding_table):
    flat_idx = idx.reshape(-1).astype(jnp.int32)
    flat_tgt = targets.reshape(-1).astype(jnp.int32)
    lse = _lse_table(token_embedding_table)
    flat_logits, partials = _sc_main(
        token_embedding_table, flat_idx, flat_tgt, lse)
    loss = jnp.sum(partials) / TOKENS
    return (flat_logits, loss)


# speedup vs baseline: 1.4705x; 1.0514x over previous
import functools

import jax
import jax.numpy as jnp
from jax import lax
from jax.experimental import pallas as pl
from jax.experimental.pallas import tpu as pltpu
from jax.experimental.pallas import tpu_sc as plsc

VOCAB = 1000
TOKENS = 1024 * 50

NC = 2
NS = 16
NW = NC * NS
L = 16
TPW = TOKENS // NW
NB = 32
NBLK = TPW // NB
GPB = NB // L


def _lse_body(t_ref, o_ref):
    t = t_ref[...]
    m = jnp.max(t, axis=1, keepdims=True)
    s = jnp.sum(jnp.exp(t - m), axis=1)
    o_ref[...] = m[:, 0] + jnp.log(s)


def _lse_table(table):
    return pl.pallas_call(
        _lse_body,
        out_shape=jax.ShapeDtypeStruct((VOCAB,), jnp.float32),
    )(table)


def _sc_body(table, idxs, tgts, lse, out, partials,
             idx_v, tgt_v, lse_v, rows0, rows1, acc_v, g0, g1, o0, o1):
    wid = lax.axis_index("s") * NC + lax.axis_index("c")
    base = pl.multiple_of(wid * TPW, 8)
    pltpu.sync_copy(idxs.at[pl.ds(base, TPW)], idx_v)
    pltpu.sync_copy(tgts.at[pl.ds(base, TPW)], tgt_v)
    pltpu.sync_copy(lse, lse_v)

    rows = (rows0, rows1)
    gs = (g0, g1)
    os_ = (o0, o1)

    def g_desc(b, buf):
        src = table.at[idx_v.at[pl.ds(pl.multiple_of(b * NB, 8), NB)]]
        return src, rows[buf], gs[buf]

    def s_desc(b, buf):
        dst = out.at[pl.ds(pl.multiple_of(base + b * NB, 8), NB)]
        return rows[buf], dst, os_[buf]

    def loss_groups(b, buf, acc):
        boff = pl.multiple_of(b * NB, 8)
        for g in range(GPB):
            off = pl.multiple_of(boff + g * L, 8)
            i16 = idx_v[pl.ds(off, L)]
            t16 = tgt_v[pl.ds(off, L)]
            l16 = plsc.load_gather(lse_v, [i16])
            rid = lax.iota(jnp.int32, L) + g * L
            p16 = plsc.load_gather(rows[buf], [rid, t16])
            acc = acc + (l16 - p16)
        return acc

    pltpu.async_copy(*g_desc(0, 0))
    pltpu.async_copy(*g_desc(1, 1))

    pltpu.make_async_copy(*g_desc(0, 0)).wait()
    acc = loss_groups(0, 0, jnp.zeros((L,), jnp.float32))
    pltpu.async_copy(*s_desc(0, 0))

    def pair(j, acc):
        for k in range(2):
            b = 2 * j + 1 + k
            buf = 1 - k
            pltpu.make_async_copy(*g_desc(b, buf)).wait()
            acc = loss_groups(b, buf, acc)
            pltpu.make_async_copy(*s_desc(b - 1, 1 - buf)).wait()
            pltpu.async_copy(*g_desc(b + 1, 1 - buf))
            pltpu.async_copy(*s_desc(b, buf))
        return acc

    acc = lax.fori_loop(0, (NBLK - 2) // 2, pair, acc)

    b = NBLK - 1
    pltpu.make_async_copy(*g_desc(b, 1)).wait()
    acc = loss_groups(b, 1, acc)
    pltpu.make_async_copy(*s_desc(b - 1, 0)).wait()
    pltpu.async_copy(*s_desc(b, 1))
    pltpu.make_async_copy(*s_desc(b, 1)).wait()

    acc_v[...] = acc
    pltpu.sync_copy(acc_v, partials.at[wid])


def _sc_main(table, flat_idx, flat_tgt, lse):
    mesh = plsc.VectorSubcoreMesh(core_axis_name="c", subcore_axis_name="s")
    f = functools.partial(
        pl.kernel,
        mesh=mesh,
        compiler_params=pltpu.CompilerParams(
            needs_layout_passes=False, use_tc_tiling_on_sc=False),
        out_type=[
            jax.ShapeDtypeStruct((TOKENS, VOCAB), jnp.float32),
            jax.ShapeDtypeStruct((NW, L), jnp.float32),
        ],
        scratch_types=[
            pltpu.VMEM((TPW,), jnp.int32),
            pltpu.VMEM((TPW,), jnp.int32),
            pltpu.VMEM((VOCAB,), jnp.float32),
            pltpu.VMEM((NB, VOCAB), jnp.float32),
            pltpu.VMEM((NB, VOCAB), jnp.float32),
            pltpu.VMEM((L,), jnp.float32),
            pltpu.SemaphoreType.DMA,
            pltpu.SemaphoreType.DMA,
            pltpu.SemaphoreType.DMA,
            pltpu.SemaphoreType.DMA,
        ],
    )(_sc_body)
    return f(table, flat_idx, flat_tgt, lse)


def kernel(idx, targets, token_embedding_table):
    flat_idx = idx.reshape(-1).astype(jnp.int32)
    flat_tgt = targets.reshape(-1).astype(jnp.int32)
    lse = _lse_table(token_embedding_table)
    flat_logits, partials = _sc_main(
        token_embedding_table, flat_idx, flat_tgt, lse)
    loss = jnp.sum(partials) / TOKENS
    return (flat_logits, loss)
